# baseline (speedup 1.0000x reference)
import functools

import jax
import jax.numpy as jnp
from jax import lax
from jax.experimental import pallas as pl
from jax.experimental.pallas import tpu as pltpu
from jax.experimental.pallas import tpu_sc as plsc

NUM_GRAPHS = 512
D_FEAT = 128
N_NODES = 100000

NUM_WORKERS = 32
CHUNK = 128
CHUNKS_PER_W = 25
QUOTA_A = 3128
QUOTA_B = 3120
N_A = 20
TAIL_A = QUOTA_A - 24 * CHUNK
TAIL_B = QUOTA_B - 24 * CHUNK
NSLOT = 6
LOOKBACK = NSLOT - 1


def _sc_unpool(x_hbm, idx_hbm, out_hbm, idx_v, buf_v, x_sh, gsem, ssem):
    c = lax.axis_index("c")
    s = lax.axis_index("s")
    wid = s * 2 + c
    is_a = wid < N_A
    base = jnp.where(is_a, wid * QUOTA_A,
                     N_A * QUOTA_A + (wid - N_A) * QUOTA_B)

    @pl.when(s == 0)
    def _():
        pltpu.sync_copy(x_hbm, x_sh)

    @pl.when(is_a)
    def _():
        pltpu.sync_copy(idx_hbm.at[pl.ds(base, QUOTA_A)],
                        idx_v.at[pl.ds(0, QUOTA_A)])

    @pl.when(jnp.logical_not(is_a))
    def _():
        pltpu.sync_copy(idx_hbm.at[pl.ds(base, QUOTA_B)],
                        idx_v.at[pl.ds(0, QUOTA_B)])

    plsc.subcore_barrier()

    def copies(j, n):
        p = lax.rem(j, NSLOT)
        g = pltpu.make_async_copy(
            x_sh.at[idx_v.at[pl.ds(j * CHUNK, n)]],
            buf_v.at[p].at[pl.ds(0, n)], gsem.at[p])
        sc = pltpu.make_async_copy(
            buf_v.at[p].at[pl.ds(0, n)],
            out_hbm.at[pl.ds(base + j * CHUNK, n)], ssem.at[p])
        return g, sc

    def branched(j, fn):
        last = j == CHUNKS_PER_W - 1

        @pl.when(jnp.logical_not(last))
        def _():
            g, sc = copies(j, CHUNK)
            fn(g, sc)

        @pl.when(last & is_a)
        def _():
            g, sc = copies(j, TAIL_A)
            fn(g, sc)

        @pl.when(last & jnp.logical_not(is_a))
        def _():
            g, sc = copies(j, TAIL_B)
            fn(g, sc)

    def body(j, carry):
        @pl.when(j < CHUNKS_PER_W)
        def _():
            @pl.when(j >= NSLOT)
            def _():
                branched(j - NSLOT, lambda g, sc: sc.wait())

            branched(j, lambda g, sc: g.start())

        @pl.when(j >= LOOKBACK)
        def _():
            def wait_then_scatter(g, sc):
                g.wait()
                sc.start()
            branched(j - LOOKBACK, wait_then_scatter)

        return carry

    lax.fori_loop(0, CHUNKS_PER_W + LOOKBACK, body, 0)

    def drain(j, carry):
        branched(j, lambda g, sc: sc.wait())
        return carry

    lax.fori_loop(CHUNKS_PER_W - NSLOT, CHUNKS_PER_W, drain, 0)


@functools.partial(jax.jit, static_argnames=())
def _run(X, idx):
    kern = pl.kernel(
        _sc_unpool,
        out_type=jax.ShapeDtypeStruct((N_NODES, D_FEAT), jnp.float32),
        mesh=plsc.VectorSubcoreMesh(core_axis_name="c", subcore_axis_name="s"),
        scratch_types=[
            pltpu.VMEM((QUOTA_A,), jnp.int32),
            pltpu.VMEM((NSLOT, CHUNK, D_FEAT), jnp.float32),
            pltpu.VMEM_SHARED((NUM_GRAPHS, D_FEAT), jnp.float32),
            pltpu.SemaphoreType.DMA((NSLOT,)),
            pltpu.SemaphoreType.DMA((NSLOT,)),
        ],
    )
    return kern(X, idx)


def kernel(X, batch):
    return _run(X, batch.astype(jnp.int32))


# speedup vs baseline: 1.0940x; 1.0940x over previous
import functools

import jax
import jax.numpy as jnp
from jax import lax
from jax.experimental import pallas as pl
from jax.experimental.pallas import tpu as pltpu
from jax.experimental.pallas import tpu_sc as plsc

NUM_GRAPHS = 512
D_FEAT = 128
N_NODES = 100000

NUM_WORKERS = 32
CHUNK = 128
CHUNKS_PER_W = 26
ROWS_PER_W = CHUNK * CHUNKS_PER_W
TAIL = N_NODES % CHUNK
TAIL_W_IDX = N_NODES - (N_NODES // ROWS_PER_W) * ROWS_PER_W
NSLOT = 6
LOOKBACK = NSLOT - 1


def _sc_unpool(x_hbm, idx_hbm, out_hbm, idx_v, buf_v, x_sh, gsem, ssem):
    c = lax.axis_index("c")
    s = lax.axis_index("s")
    wid = s * 2 + c
    base = wid * ROWS_PER_W

    @pl.when(s == 0)
    def _():
        pltpu.sync_copy(x_hbm, x_sh)

    @pl.when(base + ROWS_PER_W <= N_NODES)
    def _():
        pltpu.sync_copy(idx_hbm.at[pl.ds(base, ROWS_PER_W)], idx_v)

    @pl.when((base < N_NODES) & (base + ROWS_PER_W > N_NODES))
    def _():
        pltpu.sync_copy(idx_hbm.at[pl.ds(base, TAIL_W_IDX)],
                        idx_v.at[pl.ds(0, TAIL_W_IDX)])

    plsc.subcore_barrier()

    def row_base(j):
        return base + j * CHUNK

    def slot(j):
        return lax.rem(j, NSLOT)

    def gather(j):
        p = slot(j)
        return pltpu.make_async_copy(
            x_sh.at[idx_v.at[pl.ds(j * CHUNK, CHUNK)]],
            buf_v.at[p], gsem.at[p])

    def gather_tail(j):
        p = slot(j)
        return pltpu.make_async_copy(
            x_sh.at[idx_v.at[pl.ds(j * CHUNK, TAIL)]],
            buf_v.at[p].at[pl.ds(0, TAIL)], gsem.at[p])

    def scatter(j):
        p = slot(j)
        return pltpu.make_async_copy(
            buf_v.at[p], out_hbm.at[pl.ds(row_base(j), CHUNK)], ssem.at[p])

    def scatter_tail(j):
        p = slot(j)
        return pltpu.make_async_copy(
            buf_v.at[p].at[pl.ds(0, TAIL)],
            out_hbm.at[pl.ds(row_base(j), TAIL)], ssem.at[p])

    def full_chunk(j):
        return row_base(j) + CHUNK <= N_NODES

    def tail_chunk(j):
        r = row_base(j)
        return (r < N_NODES) & (r + CHUNK > N_NODES)

    def start_gather(j):
        @pl.when(full_chunk(j))
        def _():
            gather(j).start()

        @pl.when(tail_chunk(j))
        def _():
            gather_tail(j).start()

    def do_scatter(j):
        @pl.when(full_chunk(j))
        def _():
            gather(j).wait()
            scatter(j).start()

        @pl.when(tail_chunk(j))
        def _():
            gather_tail(j).wait()
            scatter_tail(j).start()

    def wait_scatter(j):
        @pl.when(full_chunk(j))
        def _():
            scatter(j).wait()

        @pl.when(tail_chunk(j))
        def _():
            scatter_tail(j).wait()

    def body(j, carry):
        @pl.when(j < CHUNKS_PER_W)
        def _():
            @pl.when(j >= NSLOT)
            def _():
                wait_scatter(j - NSLOT)

            start_gather(j)

        @pl.when(j >= LOOKBACK)
        def _():
            do_scatter(j - LOOKBACK)

        return carry

    lax.fori_loop(0, CHUNKS_PER_W + LOOKBACK, body, 0)

    def drain(j, carry):
        wait_scatter(j)
        return carry

    lax.fori_loop(CHUNKS_PER_W - NSLOT, CHUNKS_PER_W, drain, 0)


@functools.partial(jax.jit, static_argnames=())
def _run(X, idx):
    kern = pl.kernel(
        _sc_unpool,
        out_type=jax.ShapeDtypeStruct((N_NODES, D_FEAT), jnp.float32),
        mesh=plsc.VectorSubcoreMesh(core_axis_name="c", subcore_axis_name="s"),
        scratch_types=[
            pltpu.VMEM((ROWS_PER_W,), jnp.int32),
            pltpu.VMEM((NSLOT, CHUNK, D_FEAT), jnp.float32),
            pltpu.VMEM_SHARED((NUM_GRAPHS, D_FEAT), jnp.float32),
            pltpu.SemaphoreType.DMA((NSLOT,)),
            pltpu.SemaphoreType.DMA((NSLOT,)),
        ],
    )
    return kern(X, idx)


def kernel(X, batch):
    return _run(X, batch.astype(jnp.int32))
